# baseline (speedup 1.0000x reference)
import jax
import jax.numpy as jnp
from jax.experimental import pallas as pl
from jax.experimental.pallas import tpu as pltpu

_ROWS = 1024
_COLS = 27278
_BLOCK_ROWS = 16
_STREAMS = 4
_GRID = _ROWS // (_BLOCK_ROWS * _STREAMS)


def _mse_block_kernel(*refs):
    in_refs = refs[:2 * _STREAMS]
    sum_ref, cnt_ref = refs[2 * _STREAMS], refs[2 * _STREAMS + 1]
    s = jnp.zeros((), jnp.float32)
    c = jnp.zeros((), jnp.float32)
    for j in range(_STREAMS):
        o = in_refs[j][...]
        t = in_refs[_STREAMS + j][...]
        mask = t != -1.0
        d = o - t
        s += jnp.sum(jnp.where(mask, d * d, 0.0))
        c += jnp.sum(mask.astype(jnp.float32))
    sum_ref[...] = s.reshape(1, 1, 1)
    cnt_ref[...] = c.reshape(1, 1, 1)


def kernel(output, target):
    in_specs = []
    for _arr in range(2):
        for j in range(_STREAMS):
            in_specs.append(
                pl.BlockSpec((_BLOCK_ROWS, _COLS), lambda i, j=j: (j * _GRID + i, 0))
            )
    out_spec = pl.BlockSpec((1, 1, 1), lambda i: (i, 0, 0))
    partial_sums, partial_cnts = pl.pallas_call(
        _mse_block_kernel,
        grid=(_GRID,),
        in_specs=in_specs,
        out_specs=[out_spec, out_spec],
        out_shape=[
            jax.ShapeDtypeStruct((_GRID, 1, 1), jnp.float32),
            jax.ShapeDtypeStruct((_GRID, 1, 1), jnp.float32),
        ],
        compiler_params=pltpu.CompilerParams(
            dimension_semantics=("arbitrary",),
        ),
    )(*([output] * _STREAMS + [target] * _STREAMS))
    return jnp.sum(partial_sums) / jnp.sum(partial_cnts)


# speedup vs baseline: 1.0152x; 1.0152x over previous
import jax
import jax.numpy as jnp
from jax import lax
from jax.experimental import pallas as pl
from jax.experimental.pallas import tpu as pltpu

_ROWS = 1024
_COLS = 27278
_CH = 16
_NCHUNK = _ROWS // _CH
_NBUF = 8


def _copy(hbm_ref, buf_ref, sems, k, slot, which):
    return pltpu.make_async_copy(
        hbm_ref.at[pl.ds(k * _CH, _CH), :],
        buf_ref.at[slot],
        sems.at[which, slot],
    )


def _mse_kernel(o_hbm, t_hbm, sum_ref, cnt_ref, o_buf, t_buf, sems):
    for b in range(_NBUF):
        _copy(o_hbm, o_buf, sems, b, b, 0).start()
        _copy(t_hbm, t_buf, sems, b, b, 1).start()

    def step(k, carry):
        s, c = carry
        slot = lax.rem(k, _NBUF)
        _copy(o_hbm, o_buf, sems, k, slot, 0).wait()
        _copy(t_hbm, t_buf, sems, k, slot, 1).wait()
        o = o_buf[slot]
        t = t_buf[slot]
        mask = t != -1.0
        d = o - t
        s = s + jnp.sum(jnp.where(mask, d * d, 0.0))
        c = c + jnp.sum(mask.astype(jnp.float32))

        @pl.when(k + _NBUF < _NCHUNK)
        def _refill():
            _copy(o_hbm, o_buf, sems, k + _NBUF, slot, 0).start()
            _copy(t_hbm, t_buf, sems, k + _NBUF, slot, 1).start()

        return s, c

    s, c = lax.fori_loop(0, _NCHUNK, step, (jnp.float32(0.0), jnp.float32(0.0)))
    sum_ref[0] = s
    cnt_ref[0] = c


def kernel(output, target):
    loss_sum, n_obs = pl.pallas_call(
        _mse_kernel,
        in_specs=[
            pl.BlockSpec(memory_space=pltpu.MemorySpace.HBM),
            pl.BlockSpec(memory_space=pltpu.MemorySpace.HBM),
        ],
        out_specs=[
            pl.BlockSpec(memory_space=pltpu.MemorySpace.SMEM),
            pl.BlockSpec(memory_space=pltpu.MemorySpace.SMEM),
        ],
        out_shape=[
            jax.ShapeDtypeStruct((1,), jnp.float32),
            jax.ShapeDtypeStruct((1,), jnp.float32),
        ],
        scratch_shapes=[
            pltpu.VMEM((_NBUF, _CH, _COLS), jnp.float32),
            pltpu.VMEM((_NBUF, _CH, _COLS), jnp.float32),
            pltpu.SemaphoreType.DMA((2, _NBUF)),
        ],
    )(output, target)
    return loss_sum[0] / n_obs[0]


# speedup vs baseline: 1.0171x; 1.0019x over previous
import jax
import jax.numpy as jnp
from jax import lax
from jax.experimental import pallas as pl
from jax.experimental.pallas import tpu as pltpu

_ROWS = 1024
_COLS = 27278
_CH = 16
_NCHUNK = _ROWS // _CH
_NBUF = 8


def _copy(hbm_ref, buf_ref, sems, k, slot, which):
    return pltpu.make_async_copy(
        hbm_ref.at[pl.ds(k * _CH, _CH), :],
        buf_ref.at[slot],
        sems.at[which, slot],
    )


def _mse_kernel(o_hbm, t_hbm, sum_ref, cnt_ref, o_buf, t_buf, sems):
    for b in range(_NBUF):
        _copy(o_hbm, o_buf, sems, b, b, 0).start(priority=0)
        _copy(t_hbm, t_buf, sems, b, b, 1).start(priority=1)

    def step(k, carry):
        s, c = carry
        slot = lax.rem(k, _NBUF)
        _copy(o_hbm, o_buf, sems, k, slot, 0).wait()
        _copy(t_hbm, t_buf, sems, k, slot, 1).wait()
        o = o_buf[slot]
        t = t_buf[slot]
        mask = t != -1.0
        d = o - t
        s = s + jnp.sum(jnp.where(mask, d * d, 0.0))
        c = c + jnp.sum(mask.astype(jnp.float32))

        @pl.when(k + _NBUF < _NCHUNK)
        def _refill():
            _copy(o_hbm, o_buf, sems, k + _NBUF, slot, 0).start(priority=0)
            _copy(t_hbm, t_buf, sems, k + _NBUF, slot, 1).start(priority=1)

        return s, c

    s, c = lax.fori_loop(0, _NCHUNK, step, (jnp.float32(0.0), jnp.float32(0.0)))
    sum_ref[0] = s
    cnt_ref[0] = c


def kernel(output, target):
    loss_sum, n_obs = pl.pallas_call(
        _mse_kernel,
        in_specs=[
            pl.BlockSpec(memory_space=pltpu.MemorySpace.HBM),
            pl.BlockSpec(memory_space=pltpu.MemorySpace.HBM),
        ],
        out_specs=[
            pl.BlockSpec(memory_space=pltpu.MemorySpace.SMEM),
            pl.BlockSpec(memory_space=pltpu.MemorySpace.SMEM),
        ],
        out_shape=[
            jax.ShapeDtypeStruct((1,), jnp.float32),
            jax.ShapeDtypeStruct((1,), jnp.float32),
        ],
        scratch_shapes=[
            pltpu.VMEM((_NBUF, _CH, _COLS), jnp.float32),
            pltpu.VMEM((_NBUF, _CH, _COLS), jnp.float32),
            pltpu.SemaphoreType.DMA((2, _NBUF)),
        ],
    )(output, target)
    return loss_sum[0] / n_obs[0]
